# Initial kernel scaffold; baseline (speedup 1.0000x reference)
import jax
import jax.numpy as jnp
from jax.experimental import pallas as pl


def kernel(hidden_states, router_weight, e_bias, gate_w, up_w, down_w):
    raise NotImplementedError("write your pallas kernel here")


# speedup vs baseline: 3.4525x; 3.4525x over previous
import functools

import jax
import jax.numpy as jnp
from jax.experimental import pallas as pl
from jax.experimental.pallas import tpu as pltpu

B = 128
HIDDEN = 1024
NUM_EXPERTS = 64
TOP_K = 8
INTER = 512
NORM_MIN = 1e-12


def _moe_kernel(x_ref, rw_ref, bias_ref, gate_ref, up_ref, down_ref,
                out_ref, logits_ref, comb_ref):
    e = pl.program_id(0)

    @pl.when(e == 0)
    def _router():
        x = x_ref[...]
        logits = jnp.dot(x, rw_ref[...].T, preferred_element_type=jnp.float32)
        logits_ref[...] = logits
        probs = jax.nn.softmax(logits, axis=-1)
        scores = probs + bias_ref[...]
        col = jax.lax.broadcasted_iota(jnp.int32, scores.shape, 1)
        work = scores
        comb = jnp.zeros_like(probs)
        for _ in range(TOP_K):
            m = jnp.max(work, axis=-1, keepdims=True)
            first = jnp.min(jnp.where(work == m, col, NUM_EXPERTS),
                            axis=-1, keepdims=True)
            sel = col == first
            comb = comb + jnp.where(sel, probs, 0.0)
            work = jnp.where(sel, -jnp.inf, work)
        denom = jnp.maximum(jnp.sum(comb, axis=-1, keepdims=True), NORM_MIN)
        comb_ref[...] = comb / denom
        out_ref[...] = jnp.zeros_like(out_ref)

    x = x_ref[...]
    h = jax.nn.silu(jnp.dot(x, gate_ref[0], preferred_element_type=jnp.float32))
    h = h * jnp.dot(x, up_ref[0], preferred_element_type=jnp.float32)
    y = jnp.dot(h, down_ref[0], preferred_element_type=jnp.float32)
    ecol = jax.lax.broadcasted_iota(jnp.int32, (B, NUM_EXPERTS), 1)
    w = jnp.sum(jnp.where(ecol == e, comb_ref[...], 0.0), axis=-1, keepdims=True)
    out_ref[...] += y * w


@functools.partial(jax.jit, static_argnames=("interpret",))
def kernel(hidden_states, router_weight, e_bias, gate_w, up_w, down_w,
           interpret=False):
    shape = hidden_states.shape
    x = hidden_states.reshape(-1, HIDDEN)
    out, logits = pl.pallas_call(
        _moe_kernel,
        grid=(NUM_EXPERTS,),
        in_specs=[
            pl.BlockSpec((B, HIDDEN), lambda e: (0, 0)),
            pl.BlockSpec((NUM_EXPERTS, HIDDEN), lambda e: (0, 0)),
            pl.BlockSpec((1, NUM_EXPERTS), lambda e: (0, 0)),
            pl.BlockSpec((1, HIDDEN, INTER), lambda e: (e, 0, 0)),
            pl.BlockSpec((1, HIDDEN, INTER), lambda e: (e, 0, 0)),
            pl.BlockSpec((1, INTER, HIDDEN), lambda e: (e, 0, 0)),
        ],
        out_specs=[
            pl.BlockSpec((B, HIDDEN), lambda e: (0, 0)),
            pl.BlockSpec((B, NUM_EXPERTS), lambda e: (0, 0)),
        ],
        out_shape=[
            jax.ShapeDtypeStruct((B, HIDDEN), jnp.float32),
            jax.ShapeDtypeStruct((B, NUM_EXPERTS), jnp.float32),
        ],
        scratch_shapes=[pltpu.VMEM((B, NUM_EXPERTS), jnp.float32)],
        interpret=interpret,
    )(x, router_weight, e_bias, gate_w, up_w, down_w)
    return out.reshape(shape), logits
